# baseline (speedup 1.0000x reference)
import functools

import jax
import jax.numpy as jnp
from jax import lax
from jax.experimental import pallas as pl
from jax.experimental.pallas import tpu as pltpu
from jax.experimental.pallas import tpu_sc as plsc

_L = 16
_NC = 2
_NS = 16
_NW = _NC * _NS
_B = 16384
_D = 64
_PER_W = _B // _NW
_C = 32
_NCH = _PER_W // _C


def _sin(x):
    x2 = x * x
    p = jnp.float32(1.0 / 362880.0)
    p = p * x2 + jnp.float32(-1.0 / 5040.0)
    p = p * x2 + jnp.float32(1.0 / 120.0)
    p = p * x2 + jnp.float32(-1.0 / 6.0)
    p = p * x2 + jnp.float32(1.0)
    return x * p


def _body(heads_hbm, rels_hbm, tails_hbm, years_hbm, months_hbm, days_hbm,
          ent_hbm, rel_hbm, *rest):
    tabs_hbm = rest[:18]
    out_hbm = rest[18]
    idx_v = rest[19]
    ridx_v = rest[20]
    yr_v, mo_v, dy_v = rest[21], rest[22], rest[23]
    ent_v = rest[24]
    rel_v = rest[25]
    tab_v = rest[26:44]
    out_v = rest[44]
    sem = rest[45]

    wid = lax.axis_index("s") * _NC + lax.axis_index("c")
    base_w = wid * _PER_W

    def chunk_body(ch, carry):
        base = base_w + ch * _C
        pltpu.sync_copy(heads_hbm.at[pl.ds(base, _C)], idx_v.at[pl.ds(0, _C)])
        pltpu.sync_copy(tails_hbm.at[pl.ds(base, _C)], idx_v.at[pl.ds(_C, _C)])
        pltpu.sync_copy(rels_hbm.at[pl.ds(base, _C)], ridx_v)
        pltpu.sync_copy(years_hbm.at[pl.ds(base, _C)], yr_v)
        pltpu.sync_copy(months_hbm.at[pl.ds(base, _C)], mo_v)
        pltpu.sync_copy(days_hbm.at[pl.ds(base, _C)], dy_v)
        cps = [pltpu.async_copy(ent_hbm.at[idx_v], ent_v, sem),
               pltpu.async_copy(rel_hbm.at[ridx_v], rel_v, sem)]
        for t in range(18):
            cps.append(pltpu.async_copy(tabs_hbm[t].at[idx_v], tab_v[t], sem))
        for cp in cps:
            cp.wait()

        lane_ids = lax.iota(jnp.int32, _L)
        for g in range(_C // _L):
            yv = yr_v[pl.ds(g * _L, _L)]
            mv = mo_v[pl.ds(g * _L, _L)]
            dv = dy_v[pl.ds(g * _L, _L)]

            def lane_body(l, scores, yv=yv, mv=mv, dv=dv, g=g):
                li = jnp.full((_L,), l, jnp.int32)
                yr = jnp.take_along_axis(yv, li, axis=0)
                mo = jnp.take_along_axis(mv, li, axis=0)
                dy = jnp.take_along_axis(dv, li, axis=0)
                e = g * _L + l

                def temb(row, toff, sl):
                    r = None
                    for (tf, tp, ta), tt in zip(
                            ((0, 1, 2), (3, 4, 5), (6, 7, 8)), (yr, mo, dy)):
                        v = tab_v[toff + ta][row, sl] * _sin(
                            tab_v[toff + tf][row, sl] * tt
                            + tab_v[toff + tp][row, sl])
                        r = v if r is None else r + v
                    return r

                acc = jnp.zeros((_L,), jnp.float32)
                for j in range(_D // _L):
                    sl = pl.ds(j * _L, _L)
                    a = ent_v[e, sl]
                    b = ent_v[_C + e, sl]
                    rs = rel_v[e, sl]
                    rt = rel_v[e, pl.ds(_D + j * _L, _L)]
                    th_h = temb(e, 0, sl)
                    th_t = temb(_C + e, 0, sl)
                    tt_h = temb(e, 9, sl)
                    tt_t = temb(_C + e, 9, sl)
                    acc = acc + a * b * rs
                    acc = acc + jnp.float32(0.5) * rt * (th_h * tt_t + th_t * tt_h)
                for sh in (8, 4, 2, 1):
                    perm = jnp.bitwise_xor(lane_ids, sh)
                    acc = acc + jnp.take_along_axis(acc, perm, axis=0)
                return jnp.where(lane_ids == l, acc, scores)

            scores = lax.fori_loop(0, _L, lane_body,
                                   jnp.zeros((_L,), jnp.float32))
            out_v[pl.ds(ch * _C + g * _L, _L)] = scores
        return carry

    lax.fori_loop(0, _NCH, chunk_body, 0)
    pltpu.sync_copy(out_v, out_hbm.at[pl.ds(base_w, _PER_W)])


@jax.jit
def _run(heads, rels, tails, years, months, days, ent_emb, rel_emb, *tabs):
    mesh = plsc.VectorSubcoreMesh(core_axis_name="c", subcore_axis_name="s")
    scratch = (
        [pltpu.VMEM((2 * _C,), jnp.int32),
         pltpu.VMEM((_C,), jnp.int32),
         pltpu.VMEM((_C,), jnp.float32),
         pltpu.VMEM((_C,), jnp.float32),
         pltpu.VMEM((_C,), jnp.float32),
         pltpu.VMEM((2 * _C, _D), jnp.float32),
         pltpu.VMEM((_C, 2 * _D), jnp.float32)]
        + [pltpu.VMEM((2 * _C, _D), jnp.float32) for _ in range(18)]
        + [pltpu.VMEM((_PER_W,), jnp.float32),
           pltpu.SemaphoreType.DMA]
    )
    fn = pl.kernel(
        _body,
        mesh=mesh,
        out_type=jax.ShapeDtypeStruct((_B,), jnp.float32),
        scratch_types=scratch,
        compiler_params=pltpu.CompilerParams(use_tc_tiling_on_sc=False),
    )
    return fn(heads, rels, tails, years, months, days, ent_emb, rel_emb, *tabs)


def kernel(heads, rels, tails, years, months, days, ent_emb, rel_emb,
           y_freq_h, y_phi_h, y_amps_h, m_freq_h, m_phi_h, m_amps_h,
           d_freq_h, d_phi_h, d_amps_h, y_freq_t, y_phi_t, y_amps_t,
           m_freq_t, m_phi_t, m_amps_t, d_freq_t, d_phi_t, d_amps_t):
    return _run(heads.astype(jnp.int32), rels.astype(jnp.int32),
                tails.astype(jnp.int32), years, months, days, ent_emb, rel_emb,
                y_freq_h, y_phi_h, y_amps_h, m_freq_h, m_phi_h, m_amps_h,
                d_freq_h, d_phi_h, d_amps_h, y_freq_t, y_phi_t, y_amps_t,
                m_freq_t, m_phi_t, m_amps_t, d_freq_t, d_phi_t, d_amps_t)


# speedup vs baseline: 1.1066x; 1.1066x over previous
import jax
import jax.numpy as jnp
from jax import lax
from jax.experimental import pallas as pl
from jax.experimental.pallas import tpu as pltpu
from jax.experimental.pallas import tpu_sc as plsc

_L = 16
_NC = 2
_NS = 16
_NW = _NC * _NS
_B = 16384
_D = 64
_NT = 19
_PER_W = _B // _NW
_C = 32
_NCH = _PER_W // _C


def _sin(x):
    x2 = x * x
    p = jnp.float32(-1.0 / 5040.0)
    p = p * x2 + jnp.float32(1.0 / 120.0)
    p = p * x2 + jnp.float32(-1.0 / 6.0)
    p = p * x2 + jnp.float32(1.0)
    return x * p


def _body(heads_hbm, tails_hbm, rels_hbm, years_hbm, months_hbm, days_hbm,
          rel_hbm, *rest):
    tabs = rest[:_NT]
    out_hbm = rest[_NT]
    (hid_v, tid_v, rid_v, yr_v, mo_v, dy_v,
     hrows_v, trows_v, rrows_v, out_v, sem) = rest[_NT + 1:]

    wid = lax.axis_index("s") * _NC + lax.axis_index("c")
    base = wid * _PER_W

    pltpu.sync_copy(heads_hbm.at[pl.ds(base, _PER_W)], hid_v)
    pltpu.sync_copy(tails_hbm.at[pl.ds(base, _PER_W)], tid_v)
    pltpu.sync_copy(rels_hbm.at[pl.ds(base, _PER_W)], rid_v)
    pltpu.sync_copy(years_hbm.at[pl.ds(base * _L, _PER_W * _L)], yr_v)
    pltpu.sync_copy(months_hbm.at[pl.ds(base * _L, _PER_W * _L)], mo_v)
    pltpu.sync_copy(days_hbm.at[pl.ds(base * _L, _PER_W * _L)], dy_v)

    def chunk_body(c, carry):
        off = c * _C
        hidx = hid_v.at[pl.ds(off, _C)]
        tidx = tid_v.at[pl.ds(off, _C)]
        ridx = rid_v.at[pl.ds(off, _C)]
        cps = []
        for t in range(_NT):
            cps.append(pltpu.async_copy(tabs[t].at[hidx], hrows_v.at[t], sem))
        for t in range(_NT):
            cps.append(pltpu.async_copy(tabs[t].at[tidx], trows_v.at[t], sem))
        cps.append(pltpu.async_copy(rel_hbm.at[ridx], rrows_v, sem))
        for cp in cps:
            cp.wait()

        for eb in range(_C // _L):
            def elem_body(e16, carry2):
                e = eb * _L + e16
                bsl = pl.ds((off + e) * _L, _L)
                yr = yr_v[bsl]
                mo = mo_v[bsl]
                dy = dy_v[bsl]
                acc = jnp.zeros((_L,), jnp.float32)
                for dv in range(_D // _L):
                    sl = pl.ds(dv * _L, _L)

                    def temb(rows, toff):
                        r = None
                        for k, tt in enumerate((yr, mo, dy)):
                            f = rows[toff + 3 * k, e, sl]
                            ph = rows[toff + 3 * k + 1, e, sl]
                            a = rows[toff + 3 * k + 2, e, sl]
                            v = a * _sin(f * tt + ph)
                            r = v if r is None else r + v
                        return r

                    th_h = temb(hrows_v, 1)
                    tt_h = temb(hrows_v, 10)
                    th_t = temb(trows_v, 1)
                    tt_t = temb(trows_v, 10)
                    sh = hrows_v[0, e, sl]
                    st = trows_v[0, e, sl]
                    rs = rrows_v[e, sl]
                    rt = rrows_v[e, pl.ds(_D + dv * _L, _L)]
                    acc = (acc + sh * st * rs
                           + jnp.float32(0.5) * rt * (th_h * tt_t + th_t * tt_h))
                out_v[pl.ds((off + e) * _L, _L)] = acc
                return carry2

            lax.fori_loop(0, _L, elem_body, 0)
        return carry

    lax.fori_loop(0, _NCH, chunk_body, 0)
    pltpu.sync_copy(out_v, out_hbm.at[pl.ds(base * _L, _PER_W * _L)])


@jax.jit
def _run(heads, rels, tails, years, months, days, ent_emb, rel_emb, *tabs):
    mesh = plsc.VectorSubcoreMesh(core_axis_name="c", subcore_axis_name="s")
    fn = pl.kernel(
        _body,
        mesh=mesh,
        out_type=jax.ShapeDtypeStruct((_B * _L,), jnp.float32),
        scratch_types=[
            pltpu.VMEM((_PER_W,), jnp.int32),
            pltpu.VMEM((_PER_W,), jnp.int32),
            pltpu.VMEM((_PER_W,), jnp.int32),
            pltpu.VMEM((_PER_W * _L,), jnp.float32),
            pltpu.VMEM((_PER_W * _L,), jnp.float32),
            pltpu.VMEM((_PER_W * _L,), jnp.float32),
            pltpu.VMEM((_NT, _C, _D), jnp.float32),
            pltpu.VMEM((_NT, _C, _D), jnp.float32),
            pltpu.VMEM((_C, 2 * _D), jnp.float32),
            pltpu.VMEM((_PER_W * _L,), jnp.float32),
            pltpu.SemaphoreType.DMA,
        ],
        compiler_params=pltpu.CompilerParams(use_tc_tiling_on_sc=False),
    )
    yrb = jnp.repeat(years, _L)
    mob = jnp.repeat(months, _L)
    dyb = jnp.repeat(days, _L)
    part = fn(heads, tails, rels, yrb, mob, dyb, rel_emb, ent_emb, *tabs)
    return part.reshape(_B, _L).sum(axis=1)


def kernel(heads, rels, tails, years, months, days, ent_emb, rel_emb,
           y_freq_h, y_phi_h, y_amps_h, m_freq_h, m_phi_h, m_amps_h,
           d_freq_h, d_phi_h, d_amps_h, y_freq_t, y_phi_t, y_amps_t,
           m_freq_t, m_phi_t, m_amps_t, d_freq_t, d_phi_t, d_amps_t):
    return _run(heads.astype(jnp.int32), rels.astype(jnp.int32),
                tails.astype(jnp.int32), years, months, days, ent_emb, rel_emb,
                y_freq_h, y_phi_h, y_amps_h, m_freq_h, m_phi_h, m_amps_h,
                d_freq_h, d_phi_h, d_amps_h, y_freq_t, y_phi_t, y_amps_t,
                m_freq_t, m_phi_t, m_amps_t, d_freq_t, d_phi_t, d_amps_t)
